# Initial kernel scaffold; baseline (speedup 1.0000x reference)
import jax
import jax.numpy as jnp
from jax.experimental import pallas as pl


def kernel(feature_a, feature_b, edge_index, W, b, a_param):
    raise NotImplementedError("write your pallas kernel here")


# speedup vs baseline: 3.9357x; 3.9357x over previous
import functools

import jax
import jax.numpy as jnp
from jax import lax
from jax.experimental import pallas as pl
from jax.experimental.pallas import tpu as pltpu
from jax.experimental.pallas import tpu_sc as plsc

N_A = 10000
E = 320000
D = 128

N_PAD = 10240
NW = 32
CHUNK = 128
CHUNKS_PER_TILE = 80
E_PER_TILE = CHUNK * CHUNKS_PER_TILE
E_PAD = NW * E_PER_TILE
ROW_BLOCKS = N_PAD // CHUNK


def _dense_body(fb_ref, fa_ref, w_ref, b_ref, a1_ref, a2_ref,
                ne_ref, aa_ref, ab_ref):
    ne = jnp.dot(fb_ref[...], w_ref[...],
                 preferred_element_type=jnp.float32) + b_ref[...]
    ne_ref[...] = ne
    nrow = ne.shape[0]
    aa_ref[...] = jnp.sum(fa_ref[...] * a1_ref[...], axis=1).reshape(nrow // D, D)
    ab_ref[...] = jnp.sum(ne * a2_ref[...], axis=1).reshape(nrow // D, D)


def _dense_stage(fb_pad, fa_pad, W, b, a1, a2):
    rb = 1024
    ng = N_PAD // rb
    return pl.pallas_call(
        _dense_body,
        grid=(ng,),
        in_specs=[
            pl.BlockSpec((rb, D), lambda i: (i, 0)),
            pl.BlockSpec((rb, D), lambda i: (i, 0)),
            pl.BlockSpec((D, D), lambda i: (0, 0)),
            pl.BlockSpec((1, D), lambda i: (0, 0)),
            pl.BlockSpec((1, D), lambda i: (0, 0)),
            pl.BlockSpec((1, D), lambda i: (0, 0)),
        ],
        out_specs=[
            pl.BlockSpec((rb, D), lambda i: (i, 0)),
            pl.BlockSpec((rb // D, D), lambda i: (i, 0)),
            pl.BlockSpec((rb // D, D), lambda i: (i, 0)),
        ],
        out_shape=[
            jax.ShapeDtypeStruct((N_PAD, D), jnp.float32),
            jax.ShapeDtypeStruct((ROW_BLOCKS, D), jnp.float32),
            jax.ShapeDtypeStruct((ROW_BLOCKS, D), jnp.float32),
        ],
    )(fb_pad, fa_pad, W, b, a1, a2)


def _edge_body(aa_hbm, ab_hbm, src_hbm, dst_hbm, ne_hbm,
               out_hbm, rs_hbm,
               aa_v, ab_v, rs_v, src_v, dst_v, h_v, rows_v, acc_sh, sem):
    cid = lax.axis_index("c")
    sid = lax.axis_index("s")
    wid = cid * 16 + sid

    zeros16 = jnp.zeros((16,), jnp.float32)

    def _zrow(i, carry):
        for j in range(D // 16):
            rows_v[i, pl.ds(j * 16, 16)] = zeros16
        return carry
    lax.fori_loop(0, CHUNK, _zrow, 0)

    rows_per_sub = N_PAD // 16
    for k in range(rows_per_sub // CHUNK):
        pltpu.sync_copy(rows_v, acc_sh.at[pl.ds(sid * rows_per_sub + k * CHUNK, CHUNK)])

    def _zrs(i, carry):
        rs_v[pl.ds(i * 16, 16)] = zeros16
        return carry
    lax.fori_loop(0, N_PAD // 16, _zrs, 0)

    pltpu.sync_copy(aa_hbm, aa_v)
    pltpu.sync_copy(ab_hbm, ab_v)

    plsc.subcore_barrier()

    def _chunk(k, carry):
        base = wid * E_PER_TILE + k * CHUNK
        pltpu.sync_copy(src_hbm.at[pl.ds(base, CHUNK)], src_v)
        pltpu.sync_copy(dst_hbm.at[pl.ds(base, CHUNK)], dst_v)
        pltpu.async_copy(ne_hbm.at[dst_v], rows_v, sem).wait()

        for i in range(CHUNK // 16):
            si = src_v[pl.ds(i * 16, 16)]
            di = dst_v[pl.ds(i * 16, 16)]
            s = plsc.load_gather(aa_v, [si]) + plsc.load_gather(ab_v, [di])
            el = jnp.where(s > 0, s, 0.1 * (jnp.exp(s) - 1.0))
            h = jnp.exp(el)
            h_v[pl.ds(i * 16, 16)] = h
            plsc.addupdate_scatter(rs_v, [si], h)

        def _scale(e, carry):
            hv = plsc.load_gather(h_v, [jnp.full((16,), e, jnp.int32)])
            for j in range(D // 16):
                rows_v[e, pl.ds(j * 16, 16)] = rows_v[e, pl.ds(j * 16, 16)] * hv
            return carry
        lax.fori_loop(0, CHUNK, _scale, 0)

        pltpu.sync_copy(rows_v, acc_sh.at[src_v], add=True)
        return carry

    lax.fori_loop(0, CHUNKS_PER_TILE, _chunk, 0)

    plsc.subcore_barrier()

    for k in range(rows_per_sub // CHUNK):
        off = sid * rows_per_sub + k * CHUNK
        pltpu.sync_copy(acc_sh.at[pl.ds(off, CHUNK)],
                        out_hbm.at[cid, pl.ds(off, CHUNK)])
    pltpu.sync_copy(rs_v, rs_hbm.at[wid])


def _edge_stage(aa, ab, src_pad, dst_pad, ne_pad):
    mesh = plsc.VectorSubcoreMesh(core_axis_name="c", subcore_axis_name="s")
    k = pl.kernel(
        _edge_body,
        mesh=mesh,
        compiler_params=pltpu.CompilerParams(needs_layout_passes=False),
        out_type=[
            jax.ShapeDtypeStruct((2, N_PAD, D), jnp.float32),
            jax.ShapeDtypeStruct((NW, N_PAD), jnp.float32),
        ],
        scratch_types=[
            pltpu.VMEM((N_PAD,), jnp.float32),
            pltpu.VMEM((N_PAD,), jnp.float32),
            pltpu.VMEM((N_PAD,), jnp.float32),
            pltpu.VMEM((CHUNK,), jnp.int32),
            pltpu.VMEM((CHUNK,), jnp.int32),
            pltpu.VMEM((CHUNK,), jnp.float32),
            pltpu.VMEM((CHUNK, D), jnp.float32),
            pltpu.VMEM_SHARED((N_PAD, D), jnp.float32),
            pltpu.SemaphoreType.DMA,
        ],
    )
    return k(aa, ab, src_pad, dst_pad, ne_pad)


def _combine_body(parts_ref, rsp_ref, out_ref):
    p = parts_ref[0] + parts_ref[1]
    rs = jnp.sum(rsp_ref[...], axis=0)
    rs = jnp.where(rs == 0.0, 1.0, rs)
    out_ref[...] = p / rs[:, None]


def _combine_stage(parts, rs_parts):
    return pl.pallas_call(
        _combine_body,
        grid=(ROW_BLOCKS,),
        in_specs=[
            pl.BlockSpec((2, CHUNK, D), lambda i: (0, i, 0)),
            pl.BlockSpec((NW, CHUNK), lambda i: (0, i)),
        ],
        out_specs=pl.BlockSpec((CHUNK, D), lambda i: (i, 0)),
        out_shape=jax.ShapeDtypeStruct((N_PAD, D), jnp.float32),
    )(parts, rs_parts)


def kernel(feature_a, feature_b, edge_index, W, b, a_param):
    fa_pad = jnp.zeros((N_PAD, D), jnp.float32).at[:N_A].set(feature_a)
    fb_pad = jnp.zeros((N_PAD, D), jnp.float32).at[:N_A].set(feature_b)
    a1 = a_param[:D, 0].reshape(1, D)
    a2 = a_param[D:, 0].reshape(1, D)
    b2 = b.reshape(1, D)

    ei = edge_index.astype(jnp.int32)
    pad = jnp.full((E_PAD - E,), N_A, jnp.int32)
    src_pad = jnp.concatenate([ei[0], pad])
    dst_pad = jnp.concatenate([ei[1], pad])

    ne_pad, aa, ab = _dense_stage(fb_pad, fa_pad, W, b2, a1, a2)
    parts, rs_parts = _edge_stage(aa.reshape(N_PAD), ab.reshape(N_PAD),
                                  src_pad, dst_pad, ne_pad)
    out = _combine_stage(parts, rs_parts)
    return out[:N_A]
